# Initial kernel scaffold; baseline (speedup 1.0000x reference)
import jax
import jax.numpy as jnp
from jax.experimental import pallas as pl


def kernel(X, Ws1, bs1, Ws2, bs2, Wr, br, We1, be1, We2, be2):
    raise NotImplementedError("write your pallas kernel here")


# speedup vs baseline: 1.0886x; 1.0886x over previous
import functools
import jax
import jax.numpy as jnp
from jax.experimental import pallas as pl
from jax.experimental.pallas import tpu as pltpu

DIM = 1024
SEG = 4
SEG_DIM = DIM // SEG
E = 8
TOPK = 2
HID = 4 * DIM
EHID = 4 * SEG_DIM
NSH = 1
T = 2048

TT = 256
NTT = T // TT
KH = 1024
NKH = HID // KH


def _router_body(x_ref, wr_ref, br_ref, tw_ref, ti_ref, aux_ref, acc_ref):
    tt = pl.program_id(0)
    iota8 = jax.lax.broadcasted_iota(jnp.int32, (TT, E), 1)
    w_cols = []
    i_cols = []
    cnt_rows = []
    sw_rows = []
    for s in range(SEG):
        x = x_ref[:, s * SEG_DIM:(s + 1) * SEG_DIM]
        logits = jnp.dot(x, wr_ref[...], preferred_element_type=jnp.float32)
        logits = logits + br_ref[...]
        m1 = jnp.max(logits, axis=1, keepdims=True)
        p = jnp.exp(logits - m1)
        p = p / jnp.sum(p, axis=1, keepdims=True)
        p1 = jnp.max(p, axis=1, keepdims=True)
        i1 = jnp.min(jnp.where(p == p1, iota8, E), axis=1, keepdims=True)
        p_m = jnp.where(iota8 == i1, -jnp.inf, p)
        p2 = jnp.max(p_m, axis=1, keepdims=True)
        i2 = jnp.min(jnp.where(p_m == p2, iota8, E), axis=1, keepdims=True)
        w_cols += [p1, p2]
        i_cols += [i1, i2]
        oh1 = (iota8 == i1).astype(jnp.float32)
        oh2 = (iota8 == i2).astype(jnp.float32)
        cnt_rows.append(jnp.sum(oh1 + oh2, axis=0, keepdims=True))
        sw_rows.append(jnp.sum(oh1 * p1 + oh2 * p2, axis=0, keepdims=True))
    tw_ref[...] = jnp.concatenate(w_cols, axis=1)
    ti_ref[...] = jnp.concatenate(i_cols, axis=1)
    cnt = cnt_rows[0] + cnt_rows[1] + cnt_rows[2] + cnt_rows[3]
    sw = sw_rows[0] + sw_rows[1] + sw_rows[2] + sw_rows[3]
    part = jnp.concatenate([cnt, sw], axis=0)

    @pl.when(tt == 0)
    def _():
        acc_ref[...] = part

    @pl.when(tt > 0)
    def _():
        acc_ref[...] = acc_ref[...] + part

    @pl.when(tt == pl.num_programs(0) - 1)
    def _():
        n_tok = jnp.float32(SEG * T)
        a = acc_ref[...]
        f = a[0:1, :] / n_tok
        p_mean = a[1:2, :] / n_tok
        aux_ref[...] = jnp.float32(E) * jnp.sum(f * p_mean, keepdims=True).reshape(1, 1)


def _shared_body(x_ref, w1_ref, b1_ref, w2_ref, b2_ref, out_ref):
    k = pl.program_id(1)
    h = jnp.dot(x_ref[...], w1_ref[...], preferred_element_type=jnp.float32)
    h = jnp.maximum(h + b1_ref[...], 0.0)
    contrib = jnp.dot(h, w2_ref[...], preferred_element_type=jnp.float32)

    @pl.when(k == 0)
    def _():
        out_ref[...] = contrib + b2_ref[...]

    @pl.when(k > 0)
    def _():
        out_ref[...] = out_ref[...] + contrib


def _routed_body(x_ref, tw_ref, ti_ref, sh_ref, we1_ref, be1_ref, we2_ref,
                 be2_ref, out_ref):
    s = pl.program_id(0)
    iota8 = jax.lax.broadcasted_iota(jnp.int32, (TT, E), 1)
    seg_mask = (iota8 // TOPK) == s
    tw = jnp.where(seg_mask, tw_ref[...], 0.0)
    ti = ti_ref[...]
    x = x_ref[...]
    y = sh_ref[...]
    for e in range(E):
        w_e = jnp.sum(jnp.where(ti == e, tw, 0.0), axis=1, keepdims=True)
        h = jnp.dot(x, we1_ref[e], preferred_element_type=jnp.float32)
        h = jnp.maximum(h + be1_ref[e:e + 1, :], 0.0)
        o = jnp.dot(h, we2_ref[e], preferred_element_type=jnp.float32)
        o = o + be2_ref[e:e + 1, :]
        y = y + w_e * o
    out_ref[...] = y


@jax.jit
def kernel(X, Ws1, bs1, Ws2, bs2, Wr, br, We1, be1, We2, be2):
    b, t, c = X.shape
    X2 = X.reshape(t, c)
    br2 = br.reshape(1, E)

    top_w, top_idx, aux = pl.pallas_call(
        _router_body,
        grid=(NTT,),
        in_specs=[
            pl.BlockSpec((TT, DIM), lambda tt: (tt, 0)),
            pl.BlockSpec((SEG_DIM, E), lambda tt: (0, 0)),
            pl.BlockSpec((1, E), lambda tt: (0, 0)),
        ],
        out_specs=[
            pl.BlockSpec((TT, SEG * TOPK), lambda tt: (tt, 0)),
            pl.BlockSpec((TT, SEG * TOPK), lambda tt: (tt, 0)),
            pl.BlockSpec((1, 1), lambda tt: (0, 0)),
        ],
        out_shape=[
            jax.ShapeDtypeStruct((T, SEG * TOPK), jnp.float32),
            jax.ShapeDtypeStruct((T, SEG * TOPK), jnp.int32),
            jax.ShapeDtypeStruct((1, 1), jnp.float32),
        ],
        scratch_shapes=[pltpu.VMEM((2, E), jnp.float32)],
    )(X2, Wr, br2)

    shared = pl.pallas_call(
        _shared_body,
        grid=(NTT, NKH),
        in_specs=[
            pl.BlockSpec((TT, DIM), lambda tt, k: (tt, 0)),
            pl.BlockSpec((DIM, KH), lambda tt, k: (0, k)),
            pl.BlockSpec((1, KH), lambda tt, k: (0, k)),
            pl.BlockSpec((KH, DIM), lambda tt, k: (k, 0)),
            pl.BlockSpec((1, DIM), lambda tt, k: (0, 0)),
        ],
        out_specs=pl.BlockSpec((TT, DIM), lambda tt, k: (tt, 0)),
        out_shape=jax.ShapeDtypeStruct((T, DIM), jnp.float32),
    )(X2, Ws1[0], bs1.reshape(NSH, 1, HID)[0], Ws2[0],
      bs2.reshape(NSH, 1, DIM)[0])

    out2 = pl.pallas_call(
        _routed_body,
        grid=(SEG, NTT),
        in_specs=[
            pl.BlockSpec((TT, SEG_DIM), lambda s, tt: (tt, s)),
            pl.BlockSpec((TT, SEG * TOPK), lambda s, tt: (tt, 0)),
            pl.BlockSpec((TT, SEG * TOPK), lambda s, tt: (tt, 0)),
            pl.BlockSpec((TT, SEG_DIM), lambda s, tt: (tt, s)),
            pl.BlockSpec((E, SEG_DIM, EHID), lambda s, tt: (0, 0, 0)),
            pl.BlockSpec((E, EHID), lambda s, tt: (0, 0)),
            pl.BlockSpec((E, EHID, SEG_DIM), lambda s, tt: (0, 0, 0)),
            pl.BlockSpec((E, SEG_DIM), lambda s, tt: (0, 0)),
        ],
        out_specs=pl.BlockSpec((TT, SEG_DIM), lambda s, tt: (tt, s)),
        out_shape=jax.ShapeDtypeStruct((T, DIM), jnp.float32),
    )(X2, top_w, top_idx, shared, We1, be1, We2, be2)

    return (out2.reshape(b, t, c), aux[0, 0])
